# baseline (speedup 1.0000x reference)
import functools
import math

import jax
import jax.numpy as jnp
from jax import lax
from jax.experimental import pallas as pl
from jax.experimental.pallas import tpu as pltpu
from jax.experimental.pallas import tpu_sc as plsc

D_MODEL = 64
LANES = 16
SUB = 100


@functools.cache
def _build(B0: int, B1: int, V: int, D: int):
    info = plsc.get_sparse_core_info()
    nc, ns = info.num_cores, info.num_subcores
    nw = nc * ns
    rows_per_w = B0 // nw
    scale = math.sqrt(D)
    orows = B1 * D // 1600
    per_j = SUB * D // 1600

    mesh = plsc.VectorSubcoreMesh(core_axis_name="c", subcore_axis_name="s")

    @functools.partial(
        pl.kernel,
        out_type=jax.ShapeDtypeStruct((B0, orows, 1600), jnp.float32),
        mesh=mesh,
        scratch_types=(
            [pltpu.VMEM((rows_per_w, SUB), jnp.int32) for _ in range(4)]
            + [pltpu.VMEM((SUB, 128), jnp.float32) for _ in range(2)]
            + [pltpu.VMEM((orows, 1600), jnp.float32) for _ in range(2)]
            + [pltpu.SemaphoreType.DMA for _ in range(4)]
        ),
        compiler_params=pltpu.CompilerParams(
            use_tc_tiling_on_sc=True, needs_layout_passes=False
        ),
    )
    def emb(p0_hbm, p1_hbm, o0_hbm, o1_hbm, tbl_hbm, out_hbm, *scratch):
        pair_hbm = (p0_hbm, p1_hbm)
        off_hbm = (o0_hbm, o1_hbm)
        pair_v = scratch[0:2]
        off_v = scratch[2:4]
        gbuf = scratch[4:6]
        obuf = scratch[6:8]
        gsem = scratch[8:10]
        osem = scratch[10:12]

        wid = lax.axis_index("s") * nc + lax.axis_index("c")
        row0 = wid * rows_per_w
        for h in range(2):
            pltpu.sync_copy(
                pair_hbm[h].at[pl.ds(row0, rows_per_w)], pair_v[h]
            )
            pltpu.sync_copy(
                off_hbm[h].at[pl.ds(row0, rows_per_w)], off_v[h]
            )

        def start_gather(r, h):
            pltpu.async_copy(
                tbl_hbm.at[pair_v[h].at[r]], gbuf[h], gsem[h]
            )

        def wait_gather(r, h):
            pltpu.make_async_copy(
                tbl_hbm.at[pair_v[h].at[r]], gbuf[h], gsem[h]
            ).wait()

        def wait_out(ob):
            pltpu.make_async_copy(
                obuf[ob], out_hbm.at[0], osem[ob]
            ).wait()

        iota16 = lax.iota(jnp.int32, 16)

        for h in range(2):
            start_gather(0, h)

        def outer(i, carry):
            for rr in range(2):
                r = 2 * i + rr

                @pl.when(r >= 2)
                def _():
                    wait_out(rr)

                for h in range(2):
                    s = 2 * r + h
                    wait_gather(r, h)

                    def sel_j(j, _gb=gbuf[h], _ob=obuf[rr], _off=off_v[h],
                              _r=r, _h=h):
                        for a in range(per_j):
                            k = 25 * a + j
                            base = (k >> 4) << 4
                            lane = k & 15
                            offs = _off[_r, pl.ds(base, LANES)]
                            soff = lax.gather(
                                offs,
                                jnp.full((LANES, 1), lane, jnp.int32),
                                lax.GatherDimensionNumbers(
                                    offset_dims=(),
                                    collapsed_slice_dims=(0,),
                                    start_index_map=(0,),
                                ),
                                (1,),
                                mode=lax.GatherScatterMode.PROMISE_IN_BOUNDS,
                            )
                            row16 = jnp.full((LANES,), k, jnp.int32)
                            for c in range(D // LANES):
                                col16 = soff + (iota16 + c * LANES)
                                val = plsc.load_gather(
                                    _gb, [row16, col16]
                                )
                                _ob[
                                    per_j * _h + a,
                                    pl.ds(j * D + c * LANES, LANES),
                                ] = val * scale

                    plsc.parallel_loop(0, 1600 // D, 1, unroll=2)(sel_j)

                    @pl.when(r + 1 < rows_per_w)
                    def _():
                        start_gather(r + 1, h)

                pltpu.async_copy(
                    obuf[rr], out_hbm.at[row0 + r], osem[rr]
                )
            return carry

        lax.fori_loop(0, rows_per_w // 2, outer, 0)

        for ob in range(2):
            wait_out(ob)

    return emb


def kernel(x, table):
    b0, b1 = x.shape
    V, D = table.shape
    xi = x.astype(jnp.int32)
    xp = xi >> 1
    xo = (xi & 1) * D
    out = _build(b0, b1, V, D)(
        xp[:, :SUB],
        xp[:, SUB:],
        xo[:, :SUB],
        xo[:, SUB:],
        table.reshape(V * D // 128, 128),
    )
    return out.reshape(b0, b1, D)


# speedup vs baseline: 1.0757x; 1.0757x over previous
import functools
import math

import jax
import jax.numpy as jnp
from jax import lax
from jax.experimental import pallas as pl
from jax.experimental.pallas import tpu as pltpu
from jax.experimental.pallas import tpu_sc as plsc

D_MODEL = 64
LANES = 16
SUB = 100


@functools.cache
def _build(B0: int, B1: int, V: int, D: int):
    info = plsc.get_sparse_core_info()
    nc, ns = info.num_cores, info.num_subcores
    nw = nc * ns
    rows_per_w = B0 // nw
    scale = math.sqrt(D)
    orows = B1 * D // 1600
    per_j = SUB * D // 1600

    mesh = plsc.VectorSubcoreMesh(core_axis_name="c", subcore_axis_name="s")

    @functools.partial(
        pl.kernel,
        out_type=jax.ShapeDtypeStruct((B0, orows, 1600), jnp.float32),
        mesh=mesh,
        scratch_types=(
            [pltpu.VMEM((rows_per_w, SUB), jnp.int32) for _ in range(4)]
            + [pltpu.VMEM((SUB, 128), jnp.float32) for _ in range(2)]
            + [pltpu.VMEM((orows, 1600), jnp.float32) for _ in range(2)]
            + [pltpu.SemaphoreType.DMA for _ in range(4)]
        ),
        compiler_params=pltpu.CompilerParams(
            use_tc_tiling_on_sc=True, needs_layout_passes=False
        ),
    )
    def emb(p0_hbm, p1_hbm, o0_hbm, o1_hbm, tbl_hbm, out_hbm, *scratch):
        pair_hbm = (p0_hbm, p1_hbm)
        off_hbm = (o0_hbm, o1_hbm)
        pair_v = scratch[0:2]
        off_v = scratch[2:4]
        gbuf = scratch[4:6]
        obuf = scratch[6:8]
        gsem = scratch[8:10]
        osem = scratch[10:12]

        wid = lax.axis_index("s") * nc + lax.axis_index("c")
        row0 = wid * rows_per_w
        for h in range(2):
            pltpu.sync_copy(
                pair_hbm[h].at[pl.ds(row0, rows_per_w)], pair_v[h]
            )
            pltpu.sync_copy(
                off_hbm[h].at[pl.ds(row0, rows_per_w)], off_v[h]
            )

        def start_gather(r, h):
            pltpu.async_copy(
                tbl_hbm.at[pair_v[h].at[r]], gbuf[h], gsem[h]
            )

        def wait_gather(r, h):
            pltpu.make_async_copy(
                tbl_hbm.at[pair_v[h].at[r]], gbuf[h], gsem[h]
            ).wait()

        def wait_out(ob):
            pltpu.make_async_copy(
                obuf[ob], out_hbm.at[0], osem[ob]
            ).wait()

        iota16 = lax.iota(jnp.int32, 16)

        for h in range(2):
            start_gather(0, h)

        def outer(i, carry):
            for rr in range(2):
                r = 2 * i + rr

                @pl.when(r >= 2)
                def _():
                    wait_out(rr)

                for h in range(2):
                    s = 2 * r + h
                    wait_gather(r, h)

                    def sel_j(j, _gb=gbuf[h], _ob=obuf[rr], _off=off_v[h],
                              _r=r, _h=h):
                        for a in range(per_j):
                            k = 25 * a + j
                            base = (k >> 4) << 4
                            lane = k & 15
                            offs = _off[_r, pl.ds(base, LANES)]
                            soff = lax.gather(
                                offs,
                                jnp.full((LANES, 1), lane, jnp.int32),
                                lax.GatherDimensionNumbers(
                                    offset_dims=(),
                                    collapsed_slice_dims=(0,),
                                    start_index_map=(0,),
                                ),
                                (1,),
                                mode=lax.GatherScatterMode.PROMISE_IN_BOUNDS,
                            )
                            row16 = jnp.full((LANES,), k, jnp.int32)
                            for c in range(D // LANES):
                                col16 = soff + (iota16 + c * LANES)
                                val = plsc.load_gather(
                                    _gb, [row16, col16]
                                )
                                _ob[
                                    per_j * _h + a,
                                    pl.ds(j * D + c * LANES, LANES),
                                ] = val * scale

                    plsc.parallel_loop(0, 1600 // D, 1, unroll=1)(sel_j)

                    @pl.when(r + 1 < rows_per_w)
                    def _():
                        start_gather(r + 1, h)

                pltpu.async_copy(
                    obuf[rr], out_hbm.at[row0 + r], osem[rr]
                )
            return carry

        lax.fori_loop(0, rows_per_w // 2, outer, 0)

        for ob in range(2):
            wait_out(ob)

    return emb


def kernel(x, table):
    b0, b1 = x.shape
    V, D = table.shape
    xi = x.astype(jnp.int32)
    xp = xi >> 1
    xo = (xi & 1) * D
    out = _build(b0, b1, V, D)(
        xp[:, :SUB],
        xp[:, SUB:],
        xo[:, :SUB],
        xo[:, SUB:],
        table.reshape(V * D // 128, 128),
    )
    return out.reshape(b0, b1, D)


# speedup vs baseline: 1.1753x; 1.0925x over previous
import functools
import math

import jax
import jax.numpy as jnp
from jax import lax
from jax.experimental import pallas as pl
from jax.experimental.pallas import tpu as pltpu
from jax.experimental.pallas import tpu_sc as plsc

D_MODEL = 64
LANES = 16
SUB = 100


@functools.cache
def _build(B0: int, B1: int, V: int, D: int):
    info = plsc.get_sparse_core_info()
    nc, ns = info.num_cores, info.num_subcores
    nw = nc * ns
    rows_per_w = B0 // nw
    scale = math.sqrt(D)
    orows = B1 * D // 1600
    per_j = SUB * D // 1600

    mesh = plsc.VectorSubcoreMesh(core_axis_name="c", subcore_axis_name="s")

    @functools.partial(
        pl.kernel,
        out_type=jax.ShapeDtypeStruct((B0, orows, 1600), jnp.float32),
        mesh=mesh,
        scratch_types=(
            [pltpu.VMEM((rows_per_w, SUB), jnp.int32) for _ in range(2)]
            + [pltpu.VMEM((SUB, 128), jnp.float32) for _ in range(2)]
            + [pltpu.VMEM((orows, 1600), jnp.float32) for _ in range(2)]
            + [pltpu.SemaphoreType.DMA for _ in range(4)]
        ),
        compiler_params=pltpu.CompilerParams(
            use_tc_tiling_on_sc=True, needs_layout_passes=False
        ),
    )
    def emb(p0_hbm, p1_hbm, tbl_hbm, out_hbm, *scratch):
        pair_hbm = (p0_hbm, p1_hbm)
        pair_v = scratch[0:2]
        gbuf = scratch[2:4]
        obuf = scratch[4:6]
        gsem = scratch[6:8]
        osem = scratch[8:10]

        wid = lax.axis_index("s") * nc + lax.axis_index("c")
        row0 = wid * rows_per_w
        for h in range(2):
            pltpu.sync_copy(
                pair_hbm[h].at[pl.ds(row0, rows_per_w)], pair_v[h]
            )

        def start_gather(r, h):
            pltpu.async_copy(
                tbl_hbm.at[pair_v[h].at[r]], gbuf[h], gsem[h]
            )

        def wait_gather(r, h):
            pltpu.make_async_copy(
                tbl_hbm.at[pair_v[h].at[r]], gbuf[h], gsem[h]
            ).wait()

        def wait_out(ob):
            pltpu.make_async_copy(
                obuf[ob], out_hbm.at[0], osem[ob]
            ).wait()

        for h in range(2):
            start_gather(0, h)

        def outer(i, carry):
            for rr in range(2):
                r = 2 * i + rr

                @pl.when(r >= 2)
                def _():
                    wait_out(rr)

                for h in range(2):
                    s = 2 * r + h
                    wait_gather(r, h)

                    def sel_j(j, _gb=gbuf[h], _ob=obuf[rr], _h=h):
                        for a in range(per_j):
                            k = 25 * a + j
                            for c in range(D // LANES):
                                _ob[
                                    per_j * _h + a,
                                    pl.ds(j * D + c * LANES, LANES),
                                ] = _gb[k, pl.ds(c * LANES, LANES)] * scale

                    plsc.parallel_loop(0, 1600 // D, 1, unroll=1)(sel_j)

                    @pl.when(r + 1 < rows_per_w)
                    def _():
                        start_gather(r + 1, h)

                pltpu.async_copy(
                    obuf[rr], out_hbm.at[row0 + r], osem[rr]
                )
            return carry

        lax.fori_loop(0, rows_per_w // 2, outer, 0)

        for ob in range(2):
            wait_out(ob)

    return emb


def kernel(x, table):
    b0, b1 = x.shape
    V, D = table.shape
    xi = x.astype(jnp.int32)
    out = _build(b0, b1, V, D)(
        xi[:, :SUB],
        xi[:, SUB:],
        jnp.pad(table, ((0, 0), (0, 128 - D))),
    )
    return out.reshape(b0, b1, D)


# speedup vs baseline: 1.2139x; 1.0328x over previous
import functools
import math

import jax
import jax.numpy as jnp
from jax import lax
from jax.experimental import pallas as pl
from jax.experimental.pallas import tpu as pltpu
from jax.experimental.pallas import tpu_sc as plsc

D_MODEL = 64
LANES = 16
SUB = 100


@functools.cache
def _build(B0: int, B1: int, V: int, D: int):
    info = plsc.get_sparse_core_info()
    nc, ns = info.num_cores, info.num_subcores
    nw = nc * ns
    rows_per_w = B0 // nw
    scale = math.sqrt(D)
    orows = B1 * D // 1600
    per_j = SUB * D // 1600

    mesh = plsc.VectorSubcoreMesh(core_axis_name="c", subcore_axis_name="s")

    @functools.partial(
        pl.kernel,
        out_type=jax.ShapeDtypeStruct((B0, orows, 1600), jnp.float32),
        mesh=mesh,
        scratch_types=(
            [pltpu.VMEM((rows_per_w, SUB), jnp.int32) for _ in range(2)]
            + [pltpu.VMEM((SUB, 128), jnp.float32) for _ in range(4)]
            + [pltpu.VMEM((orows, 1600), jnp.float32) for _ in range(2)]
            + [pltpu.SemaphoreType.DMA for _ in range(6)]
        ),
        compiler_params=pltpu.CompilerParams(
            use_tc_tiling_on_sc=True, needs_layout_passes=False
        ),
    )
    def emb(p0_hbm, p1_hbm, tbl_hbm, out_hbm, *scratch):
        pair_hbm = (p0_hbm, p1_hbm)
        pair_v = scratch[0:2]
        gbuf = scratch[2:6]
        obuf = scratch[6:8]
        gsem = scratch[8:12]
        osem = scratch[12:14]

        wid = lax.axis_index("s") * nc + lax.axis_index("c")
        row0 = wid * rows_per_w
        for h in range(2):
            pltpu.sync_copy(
                pair_hbm[h].at[pl.ds(row0, rows_per_w)], pair_v[h]
            )

        def start_gather(r, h, b):
            pltpu.async_copy(
                tbl_hbm.at[pair_v[h].at[r]], gbuf[b], gsem[b]
            )

        def wait_gather(r, h, b):
            pltpu.make_async_copy(
                tbl_hbm.at[pair_v[h].at[r]], gbuf[b], gsem[b]
            ).wait()

        def wait_out(ob):
            pltpu.make_async_copy(
                obuf[ob], out_hbm.at[0], osem[ob]
            ).wait()

        for rr in range(2):
            for h in range(2):
                start_gather(rr, h, 2 * rr + h)

        def outer(i, carry):
            for rr in range(2):
                r = 2 * i + rr

                @pl.when(r >= 2)
                def _():
                    wait_out(rr)

                for h in range(2):
                    b = 2 * rr + h
                    wait_gather(r, h, b)

                    def sel_j(j, _gb=gbuf[b], _ob=obuf[rr], _h=h):
                        for a in range(per_j):
                            k = 25 * a + j
                            for c in range(D // LANES):
                                _ob[
                                    per_j * _h + a,
                                    pl.ds(j * D + c * LANES, LANES),
                                ] = _gb[k, pl.ds(c * LANES, LANES)] * scale

                    plsc.parallel_loop(0, 1600 // D, 1, unroll=1)(sel_j)

                    @pl.when(r + 2 < rows_per_w)
                    def _():
                        start_gather(r + 2, h, b)

                pltpu.async_copy(
                    obuf[rr], out_hbm.at[row0 + r], osem[rr]
                )
            return carry

        lax.fori_loop(0, rows_per_w // 2, outer, 0)

        for ob in range(2):
            wait_out(ob)

    return emb


def kernel(x, table):
    b0, b1 = x.shape
    V, D = table.shape
    xi = x.astype(jnp.int32)
    out = _build(b0, b1, V, D)(
        xi[:, :SUB],
        xi[:, SUB:],
        jnp.pad(table, ((0, 0), (0, 128 - D))),
    )
    return out.reshape(b0, b1, D)


# speedup vs baseline: 1.2141x; 1.0002x over previous
import functools
import math

import jax
import jax.numpy as jnp
from jax import lax
from jax.experimental import pallas as pl
from jax.experimental.pallas import tpu as pltpu
from jax.experimental.pallas import tpu_sc as plsc

D_MODEL = 64
LANES = 16
SUB = 100


@functools.cache
def _build(B0: int, B1: int, V: int, D: int):
    info = plsc.get_sparse_core_info()
    nc, ns = info.num_cores, info.num_subcores
    nw = nc * ns
    rows_per_w = B0 // nw
    scale = math.sqrt(D)
    orows = B1 * D // 1600
    per_j = SUB * D // 1600

    mesh = plsc.VectorSubcoreMesh(core_axis_name="c", subcore_axis_name="s")

    @functools.partial(
        pl.kernel,
        out_type=jax.ShapeDtypeStruct((B0, orows, 1600), jnp.float32),
        mesh=mesh,
        scratch_types=(
            [pltpu.VMEM((rows_per_w, SUB), jnp.int32) for _ in range(2)]
            + [pltpu.VMEM((SUB, 128), jnp.float32) for _ in range(4)]
            + [pltpu.VMEM((orows, 1600), jnp.float32) for _ in range(2)]
            + [pltpu.SemaphoreType.DMA for _ in range(6)]
        ),
        compiler_params=pltpu.CompilerParams(
            use_tc_tiling_on_sc=True, needs_layout_passes=False
        ),
    )
    def emb(i0_hbm, i1_hbm, tbl_hbm, out_hbm, *scratch):
        idx_hbm = (i0_hbm, i1_hbm)
        idx_v = scratch[0:2]
        gbuf = scratch[2:6]
        obuf = scratch[6:8]
        gsem = scratch[8:12]
        osem = scratch[12:14]

        wid = lax.axis_index("s") * nc + lax.axis_index("c")
        row0 = wid * rows_per_w
        for h in range(2):
            pltpu.sync_copy(
                idx_hbm[h].at[pl.ds(row0, rows_per_w)], idx_v[h]
            )

        def start_gather(r, h, b):
            pltpu.async_copy(
                tbl_hbm.at[idx_v[h].at[r]], gbuf[b], gsem[b]
            )

        def wait_gather(r, h, b):
            pltpu.make_async_copy(
                tbl_hbm.at[idx_v[h].at[r]], gbuf[b], gsem[b]
            ).wait()

        def wait_out(ob):
            pltpu.make_async_copy(
                obuf[ob], out_hbm.at[0], osem[ob]
            ).wait()

        for rr in range(2):
            for h in range(2):
                start_gather(rr, h, 2 * rr + h)

        def outer(i, carry):
            for rr in range(2):
                r = 2 * i + rr

                @pl.when(r >= 2)
                def _():
                    wait_out(rr)

                for h in range(2):
                    b = 2 * rr + h
                    wait_gather(r, h, b)

                    def sel_j(j, _gb=gbuf[b], _ob=obuf[rr], _h=h):
                        for a in range(per_j):
                            k = 25 * a + j
                            for c in range(D // LANES):
                                _ob[
                                    per_j * _h + a,
                                    pl.ds(j * D + c * LANES, LANES),
                                ] = _gb[k, pl.ds(c * LANES, LANES)] * scale

                    plsc.parallel_loop(0, 1600 // D, 1, unroll=1)(sel_j)

                    @pl.when(r + 2 < rows_per_w)
                    def _():
                        start_gather(r + 2, h, b)

                pltpu.async_copy(
                    obuf[rr], out_hbm.at[row0 + r], osem[rr]
                )
            return carry

        lax.fori_loop(0, rows_per_w // 2, outer, 0)

        for ob in range(2):
            wait_out(ob)

    return emb


def kernel(x, table):
    b0, b1 = x.shape
    V, D = table.shape
    xi = x.astype(jnp.int32)
    out = _build(b0, b1, V, D)(
        xi[:, :SUB],
        xi[:, SUB:],
        jnp.pad(table, ((0, 0), (0, 128 - D))),
    )
    return out.reshape(b0, b1, D)
